# baseline (speedup 1.0000x reference)
import functools

import jax
import jax.numpy as jnp
from jax import lax
from jax.experimental import pallas as pl
from jax.experimental.pallas import tpu as pltpu
from jax.experimental.pallas import tpu_sc as plsc

VOCAB = 1000000
EMBED = 64
BATCH = 4096
HIST = 200
N = BATCH * HIST

_info = plsc.get_sparse_core_info()
NC = _info.num_cores
NS = _info.num_subcores
NW = NC * NS
BPW = N // NW
K = 128
NCHUNK = BPW // K

_mesh = plsc.VectorSubcoreMesh(core_axis_name="c", subcore_axis_name="s")


@functools.partial(
    pl.kernel,
    mesh=_mesh,
    out_type=jax.ShapeDtypeStruct((N, EMBED), jnp.float32),
    compiler_params=pltpu.CompilerParams(use_tc_tiling_on_sc=False),
    scratch_types=[
        pltpu.VMEM((BPW,), jnp.int32),
        pltpu.VMEM((2, K, EMBED), jnp.float32),
        pltpu.SemaphoreType.DMA,
        pltpu.SemaphoreType.DMA,
    ],
)
def _embed(seq_hbm, table_hbm, out_hbm, idx_v, rows_v, gsem, osem):
    wid = lax.axis_index("s") * NC + lax.axis_index("c")
    base = wid * BPW
    pltpu.sync_copy(seq_hbm.at[pl.ds(base, BPW)], idx_v)

    def chunk(g, _):
        pltpu.async_copy(
            table_hbm.at[idx_v.at[pl.ds(g * K, K)]], rows_v.at[0], gsem
        ).wait()
        pltpu.sync_copy(rows_v.at[0], out_hbm.at[pl.ds(base + g * K, K)])
        return 0

    lax.fori_loop(0, NCHUNK, chunk, 0)


def kernel(seq, weight):
    out = _embed(seq.reshape(N), weight)
    return out.reshape(BATCH, HIST, EMBED)


# speedup vs baseline: 1.1161x; 1.1161x over previous
import functools

import jax
import jax.numpy as jnp
from jax import lax
from jax.experimental import pallas as pl
from jax.experimental.pallas import tpu as pltpu
from jax.experimental.pallas import tpu_sc as plsc

VOCAB = 1000000
EMBED = 64
BATCH = 4096
HIST = 200
N = BATCH * HIST

_info = plsc.get_sparse_core_info()
NC = _info.num_cores
NS = _info.num_subcores
NW = NC * NS
BPW = N // NW
K = 128
NCHUNK = BPW // K
NBUF = 4

_mesh = plsc.VectorSubcoreMesh(core_axis_name="c", subcore_axis_name="s")


@functools.partial(
    pl.kernel,
    mesh=_mesh,
    out_type=jax.ShapeDtypeStruct((N, EMBED), jnp.float32),
    compiler_params=pltpu.CompilerParams(use_tc_tiling_on_sc=False),
    scratch_types=[pltpu.VMEM((BPW,), jnp.int32),
                   pltpu.VMEM((NBUF, K, EMBED), jnp.float32)]
                  + [pltpu.SemaphoreType.DMA] * NBUF,
)
def _embed(seq_hbm, table_hbm, out_hbm, idx_v, rows_v, *sems):
    wid = lax.axis_index("s") * NC + lax.axis_index("c")
    base = wid * BPW
    pltpu.sync_copy(seq_hbm.at[pl.ds(base, BPW)], idx_v)

    def issue_gather(g, b):
        pltpu.async_copy(
            table_hbm.at[idx_v.at[pl.ds(g * K, K)]], rows_v.at[b], sems[b]
        )

    def wait_gather(b):
        pltpu.make_async_copy(
            table_hbm.at[pl.ds(0, K)], rows_v.at[b], sems[b]
        ).wait()

    def store(g, b):
        pltpu.sync_copy(rows_v.at[b], out_hbm.at[pl.ds(base + g * K, K)])

    for b in range(NBUF):
        issue_gather(b, b)

    def body(j, _):
        for b in range(NBUF):
            g = j * NBUF + b
            wait_gather(b)
            store(g, b)
            issue_gather(g + NBUF, b)
        return 0

    lax.fori_loop(0, NCHUNK // NBUF - 1, body, 0)

    for b in range(NBUF):
        g = NCHUNK - NBUF + b
        wait_gather(b)
        store(g, b)


def kernel(seq, weight):
    out = _embed(seq.reshape(N), weight)
    return out.reshape(BATCH, HIST, EMBED)


# speedup vs baseline: 1.1174x; 1.0012x over previous
import functools

import jax
import jax.numpy as jnp
from jax import lax
from jax.experimental import pallas as pl
from jax.experimental.pallas import tpu as pltpu
from jax.experimental.pallas import tpu_sc as plsc

VOCAB = 1000000
EMBED = 64
BATCH = 4096
HIST = 200
N = BATCH * HIST

_info = plsc.get_sparse_core_info()
NC = _info.num_cores
NS = _info.num_subcores
NW = NC * NS
BPW = N // NW
K = 128
NCHUNK = BPW // K
NRING = 8
LEAD = 4

_mesh = plsc.VectorSubcoreMesh(core_axis_name="c", subcore_axis_name="s")


@functools.partial(
    pl.kernel,
    mesh=_mesh,
    out_type=jax.ShapeDtypeStruct((N, EMBED), jnp.float32),
    compiler_params=pltpu.CompilerParams(use_tc_tiling_on_sc=False),
    scratch_types=[pltpu.VMEM((BPW,), jnp.int32),
                   pltpu.VMEM((NRING, K, EMBED), jnp.float32)]
                  + [pltpu.SemaphoreType.DMA] * NRING,
)
def _embed(seq_hbm, table_hbm, out_hbm, idx_v, rows_v, *sems):
    wid = lax.axis_index("s") * NC + lax.axis_index("c")
    base = wid * BPW
    pltpu.sync_copy(seq_hbm.at[pl.ds(base, BPW)], idx_v)

    def issue_gather(g, b):
        pltpu.async_copy(
            table_hbm.at[idx_v.at[pl.ds(g * K, K)]], rows_v.at[b], sems[b]
        )

    def wait_gather(b):
        pltpu.make_async_copy(
            table_hbm.at[pl.ds(0, K)], rows_v.at[b], sems[b]
        ).wait()

    def store(g, b):
        pltpu.sync_copy(rows_v.at[b], out_hbm.at[pl.ds(base + g * K, K)])

    for b in range(LEAD):
        issue_gather(b, b)

    def body(j, _):
        for i in range(NRING):
            g = j * NRING + i
            issue_gather(g + LEAD, (i + LEAD) % NRING)
            wait_gather(i)
            store(g, i)
        return 0

    lax.fori_loop(0, NCHUNK // NRING - 1, body, 0)

    for i in range(NRING):
        g = NCHUNK - NRING + i
        if g + LEAD < NCHUNK:
            issue_gather(g + LEAD, (i + LEAD) % NRING)
        wait_gather(i)
        store(g, i)


def kernel(seq, weight):
    out = _embed(seq.reshape(N), weight)
    return out.reshape(BATCH, HIST, EMBED)
